# Initial kernel scaffold; baseline (speedup 1.0000x reference)
import jax
import jax.numpy as jnp
from jax.experimental import pallas as pl


def kernel(input_tensor, token_table, segment_table, ln_gamma, ln_beta):
    raise NotImplementedError("write your pallas kernel here")


# speedup vs baseline: 2.6232x; 2.6232x over previous
import functools

import jax
import jax.numpy as jnp
import numpy as np
from jax import lax
from jax.experimental import pallas as pl
from jax.experimental.pallas import tpu as pltpu
from jax.experimental.pallas import tpu_sc as plsc

_NC = 2
_NS = 16
_NW = _NC * _NS
_CHUNK = 64


def _pos_encoding(size: int, sent: int) -> jnp.ndarray:
    pos = jnp.arange(sent, dtype=jnp.float32)[:, None]
    d = 2.0 * jnp.arange(size, dtype=jnp.float32) / float(size)
    pos = pos / (1e4 ** d)
    pos = pos.at[:, 0::2].set(jnp.sin(pos[:, 0::2]))
    pos = pos.at[:, 1::2].set(jnp.cos(pos[:, 1::2]))
    return pos


@functools.lru_cache(maxsize=None)
def _sc_gather_fn(n_rows: int, size: int):
    bpw = n_rows // _NW
    nchunk = bpw // _CHUNK
    mesh = plsc.VectorSubcoreMesh(core_axis_name="c", subcore_axis_name="s")

    @functools.partial(
        pl.kernel,
        mesh=mesh,
        out_type=jax.ShapeDtypeStruct((n_rows, size), jnp.float32),
        scratch_types=[
            pltpu.VMEM((nchunk, _CHUNK), jnp.int32),
            pltpu.VMEM((2, _CHUNK, size), jnp.float32),
            pltpu.SemaphoreType.DMA,
            pltpu.SemaphoreType.DMA,
        ],
    )
    def gather_kernel(table_hbm, idx_hbm, out_hbm, idx_v, rows_v, sem0, sem1):
        wid = lax.axis_index("s") * _NC + lax.axis_index("c")
        base = wid * bpw
        pltpu.sync_copy(idx_hbm.at[wid], idx_v)

        def start(c, buf, sem):
            return pltpu.async_copy(
                table_hbm.at[idx_v.at[c]],
                rows_v.at[buf],
                sem,
            )

        sems = (sem0, sem1)
        cps = [start(0, 0, sem0)]
        if nchunk > 1:
            cps.append(start(1, 1, sem1))
        for c in range(nchunk):
            buf = c % 2
            cps[c].wait()
            pltpu.sync_copy(
                rows_v.at[buf], out_hbm.at[pl.ds(base + c * _CHUNK, _CHUNK)]
            )
            if c + 2 < nchunk:
                cps.append(start(c + 2, buf, sems[buf]))

    return gather_kernel


@functools.lru_cache(maxsize=None)
def _tc_ln_fn(batch: int, sent: int, size: int, s_blk: int):
    seg_boundary = sent // 2 + 1
    grid = (batch, sent // s_blk)

    def body(g_ref, seg_ref, pos_ref, gam_ref, bet_ref, o_ref):
        j = pl.program_id(1)
        rows = j * s_blk + lax.broadcasted_iota(jnp.int32, (s_blk, 1), 0)
        segv = jnp.where(rows >= seg_boundary, seg_ref[1], seg_ref[0])
        x = g_ref[0] + segv + pos_ref[...]
        mean = jnp.mean(x, axis=-1, keepdims=True)
        xc = x - mean
        var = jnp.mean(xc * xc, axis=-1, keepdims=True)
        y = xc * lax.rsqrt(var + 1e-5)
        o_ref[0] = y * gam_ref[...] + bet_ref[...]

    return pl.pallas_call(
        body,
        grid=grid,
        in_specs=[
            pl.BlockSpec((1, s_blk, size), lambda b, j: (b, j, 0)),
            pl.BlockSpec((2, size), lambda b, j: (0, 0)),
            pl.BlockSpec((s_blk, size), lambda b, j: (j, 0)),
            pl.BlockSpec((1, size), lambda b, j: (0, 0)),
            pl.BlockSpec((1, size), lambda b, j: (0, 0)),
        ],
        out_specs=pl.BlockSpec((1, s_blk, size), lambda b, j: (b, j, 0)),
        out_shape=jax.ShapeDtypeStruct((batch, sent, size), jnp.float32),
    )


def kernel(input_tensor, token_table, segment_table, ln_gamma, ln_beta):
    batch, sent = input_tensor.shape
    size = token_table.shape[1]
    n_rows = batch * sent

    flat_idx = input_tensor.reshape(
        _NW, n_rows // _NW // _CHUNK, _CHUNK
    ).astype(jnp.int32)
    gathered = _sc_gather_fn(n_rows, size)(token_table, flat_idx)
    g = gathered.reshape(batch, sent, size)

    seg2 = segment_table[:2]
    pos = _pos_encoding(size, sent)
    out = _tc_ln_fn(batch, sent, size, 256)(
        g, seg2, pos, ln_gamma.reshape(1, size), ln_beta.reshape(1, size)
    )
    return out


# speedup vs baseline: 4.8594x; 1.8525x over previous
import functools

import jax
import jax.numpy as jnp
import numpy as np
from jax import lax
from jax.experimental import pallas as pl
from jax.experimental.pallas import tpu as pltpu
from jax.experimental.pallas import tpu_sc as plsc

_NC = 2
_NS = 16
_NW = _NC * _NS
_CHUNK = 64


@functools.lru_cache(maxsize=None)
def _pos_encoding(size: int, sent: int) -> np.ndarray:
    with jax.ensure_compile_time_eval():
        pos = jnp.arange(sent, dtype=jnp.float32)[:, None]
        d = 2.0 * jnp.arange(size, dtype=jnp.float32) / float(size)
        pos = pos / (1e4 ** d)
        pos = pos.at[:, 0::2].set(jnp.sin(pos[:, 0::2]))
        pos = pos.at[:, 1::2].set(jnp.cos(pos[:, 1::2]))
        return np.asarray(pos)


@functools.lru_cache(maxsize=None)
def _sc_gather_fn(n_rows: int, size: int):
    bpw = n_rows // _NW
    nchunk = bpw // _CHUNK
    mesh = plsc.VectorSubcoreMesh(core_axis_name="c", subcore_axis_name="s")

    @functools.partial(
        pl.kernel,
        mesh=mesh,
        out_type=jax.ShapeDtypeStruct((n_rows, size), jnp.float32),
        scratch_types=[
            pltpu.VMEM((nchunk, _CHUNK), jnp.int32),
            pltpu.VMEM((2, _CHUNK, size), jnp.float32),
            pltpu.SemaphoreType.DMA,
            pltpu.SemaphoreType.DMA,
        ],
    )
    def gather_kernel(table_hbm, idx_hbm, out_hbm, idx_v, rows_v, sem0, sem1):
        wid = lax.axis_index("s") * _NC + lax.axis_index("c")
        base = wid * bpw
        pltpu.sync_copy(idx_hbm.at[wid], idx_v)

        def start(c, buf, sem):
            return pltpu.async_copy(
                table_hbm.at[idx_v.at[c]],
                rows_v.at[buf],
                sem,
            )

        sems = (sem0, sem1)
        cps = [start(0, 0, sem0)]
        if nchunk > 1:
            cps.append(start(1, 1, sem1))
        for c in range(nchunk):
            buf = c % 2
            cps[c].wait()
            pltpu.sync_copy(
                rows_v.at[buf], out_hbm.at[pl.ds(base + c * _CHUNK, _CHUNK)]
            )
            if c + 2 < nchunk:
                cps.append(start(c + 2, buf, sems[buf]))

    return gather_kernel


@functools.lru_cache(maxsize=None)
def _tc_ln_fn(batch: int, sent: int, size: int, s_blk: int):
    seg_boundary = sent // 2 + 1
    grid = (sent // s_blk, batch)

    def body(g_ref, seg_ref, pos_ref, gam_ref, bet_ref, o_ref):
        j = pl.program_id(0)
        rows = j * s_blk + lax.broadcasted_iota(jnp.int32, (s_blk, 1), 0)
        segv = jnp.where(rows >= seg_boundary, seg_ref[1], seg_ref[0])
        x = g_ref[0] + segv + pos_ref[...]
        mean = jnp.mean(x, axis=-1, keepdims=True)
        xc = x - mean
        var = jnp.mean(xc * xc, axis=-1, keepdims=True)
        y = xc * lax.rsqrt(var + 1e-5)
        o_ref[0] = y * gam_ref[...] + bet_ref[...]

    return pl.pallas_call(
        body,
        grid=grid,
        in_specs=[
            pl.BlockSpec((1, s_blk, size), lambda j, b: (b, j, 0)),
            pl.BlockSpec((2, size), lambda j, b: (0, 0)),
            pl.BlockSpec((s_blk, size), lambda j, b: (j, 0)),
            pl.BlockSpec((1, size), lambda j, b: (0, 0)),
            pl.BlockSpec((1, size), lambda j, b: (0, 0)),
        ],
        out_specs=pl.BlockSpec((1, s_blk, size), lambda j, b: (b, j, 0)),
        out_shape=jax.ShapeDtypeStruct((batch, sent, size), jnp.float32),
    )


def kernel(input_tensor, token_table, segment_table, ln_gamma, ln_beta):
    batch, sent = input_tensor.shape
    size = token_table.shape[1]
    n_rows = batch * sent

    flat_idx = input_tensor.reshape(
        _NW, n_rows // _NW // _CHUNK, _CHUNK
    ).astype(jnp.int32)
    gathered = _sc_gather_fn(n_rows, size)(token_table, flat_idx)
    g = gathered.reshape(batch, sent, size)

    seg2 = segment_table[:2]
    pos = jnp.asarray(_pos_encoding(size, sent))
    out = _tc_ln_fn(batch, sent, size, 256)(
        g, seg2, pos, ln_gamma.reshape(1, size), ln_beta.reshape(1, size)
    )
    return out


# speedup vs baseline: 4.9704x; 1.0228x over previous
import functools

import jax
import jax.numpy as jnp
import numpy as np
from jax import lax
from jax.experimental import pallas as pl
from jax.experimental.pallas import tpu as pltpu
from jax.experimental.pallas import tpu_sc as plsc

_NC = 2
_NS = 16
_NW = _NC * _NS
_CHUNK = 64


@functools.lru_cache(maxsize=None)
def _pos_encoding(size: int, sent: int) -> np.ndarray:
    with jax.ensure_compile_time_eval():
        pos = jnp.arange(sent, dtype=jnp.float32)[:, None]
        d = 2.0 * jnp.arange(size, dtype=jnp.float32) / float(size)
        pos = pos / (1e4 ** d)
        pos = pos.at[:, 0::2].set(jnp.sin(pos[:, 0::2]))
        pos = pos.at[:, 1::2].set(jnp.cos(pos[:, 1::2]))
        return np.asarray(pos)


@functools.lru_cache(maxsize=None)
def _sc_gather_fn(n_rows: int, size: int):
    bpw = n_rows // _NW
    nchunk = bpw // _CHUNK
    mesh = plsc.VectorSubcoreMesh(core_axis_name="c", subcore_axis_name="s")

    @functools.partial(
        pl.kernel,
        mesh=mesh,
        out_type=jax.ShapeDtypeStruct((n_rows, size), jnp.float32),
        scratch_types=[
            pltpu.VMEM((nchunk, _CHUNK), jnp.int32),
            pltpu.VMEM((2, _CHUNK, size), jnp.float32),
            pltpu.SemaphoreType.DMA,
            pltpu.SemaphoreType.DMA,
        ],
    )
    def gather_kernel(table_hbm, idx_hbm, out_hbm, idx_v, rows_v, sem0, sem1):
        wid = lax.axis_index("s") * _NC + lax.axis_index("c")
        base = wid * bpw
        pltpu.sync_copy(idx_hbm.at[wid], idx_v)

        def start(c, buf, sem):
            return pltpu.async_copy(
                table_hbm.at[idx_v.at[c]],
                rows_v.at[buf],
                sem,
            )

        sems = (sem0, sem1)
        cps = [start(0, 0, sem0)]
        if nchunk > 1:
            cps.append(start(1, 1, sem1))
        for c in range(nchunk):
            buf = c % 2
            cps[c].wait()
            pltpu.sync_copy(
                rows_v.at[buf], out_hbm.at[pl.ds(base + c * _CHUNK, _CHUNK)]
            )
            if c + 2 < nchunk:
                cps.append(start(c + 2, buf, sems[buf]))

    return gather_kernel


@functools.lru_cache(maxsize=None)
def _tc_ln_fn(batch: int, nb: int, b0: int, sent: int, size: int, s_blk: int):
    seg_boundary = sent // 2 + 1
    grid = (sent // s_blk, nb)
    alias = b0 > 0

    def body(*refs):
        if alias:
            refs = refs[1:]
        g_ref, seg_ref, pos_ref, gam_ref, bet_ref, o_ref = refs
        j = pl.program_id(0)
        rows = j * s_blk + lax.broadcasted_iota(jnp.int32, (s_blk, 1), 0)
        segv = jnp.where(rows >= seg_boundary, seg_ref[1], seg_ref[0])
        x = g_ref[0] + segv + pos_ref[...]
        mean = jnp.mean(x, axis=-1, keepdims=True)
        xc = x - mean
        var = jnp.mean(xc * xc, axis=-1, keepdims=True)
        y = xc * lax.rsqrt(var + 1e-5)
        o_ref[0] = y * gam_ref[...] + bet_ref[...]

    in_specs = [
        pl.BlockSpec((1, s_blk, size), lambda j, b: (b, j, 0)),
        pl.BlockSpec((2, size), lambda j, b: (0, 0)),
        pl.BlockSpec((s_blk, size), lambda j, b: (j, 0)),
        pl.BlockSpec((1, size), lambda j, b: (0, 0)),
        pl.BlockSpec((1, size), lambda j, b: (0, 0)),
    ]
    kwargs = {}
    if alias:
        in_specs = [pl.BlockSpec(memory_space=pl.ANY)] + in_specs
        kwargs["input_output_aliases"] = {0: 0}

    return pl.pallas_call(
        body,
        grid=grid,
        in_specs=in_specs,
        out_specs=pl.BlockSpec((1, s_blk, size), lambda j, b: (b0 + b, j, 0)),
        out_shape=jax.ShapeDtypeStruct((batch, sent, size), jnp.float32),
        **kwargs,
    )


def kernel(input_tensor, token_table, segment_table, ln_gamma, ln_beta):
    batch, sent = input_tensor.shape
    size = token_table.shape[1]
    nsplit = 2 if batch % 2 == 0 else 1
    nb = batch // nsplit
    half_rows = nb * sent
    bpw = half_rows // _NW

    idx_all = input_tensor.reshape(
        nsplit, _NW, bpw // _CHUNK, _CHUNK
    ).astype(jnp.int32)

    gather = _sc_gather_fn(half_rows, size)
    halves = [
        gather(token_table, idx_all[h]).reshape(nb, sent, size)
        for h in range(nsplit)
    ]

    seg2 = segment_table[:2]
    pos = jnp.asarray(_pos_encoding(size, sent))
    gamma = ln_gamma.reshape(1, size)
    beta = ln_beta.reshape(1, size)

    out = _tc_ln_fn(batch, nb, 0, sent, size, 256)(
        halves[0], seg2, pos, gamma, beta
    )
    for h in range(1, nsplit):
        out = _tc_ln_fn(batch, nb, h * nb, sent, size, 256)(
            out, halves[h], seg2, pos, gamma, beta
        )
    return out


# speedup vs baseline: 5.9570x; 1.1985x over previous
import functools

import jax
import jax.numpy as jnp
import numpy as np
from jax import lax
from jax.experimental import pallas as pl
from jax.experimental.pallas import tpu as pltpu
from jax.experimental.pallas import tpu_sc as plsc

_NC = 2
_NS = 16
_NW = _NC * _NS
_CHUNK = 64


@functools.lru_cache(maxsize=None)
def _pos_encoding(size: int, sent: int) -> np.ndarray:
    with jax.ensure_compile_time_eval():
        pos = jnp.arange(sent, dtype=jnp.float32)[:, None]
        d = 2.0 * jnp.arange(size, dtype=jnp.float32) / float(size)
        pos = pos / (1e4 ** d)
        pos = pos.at[:, 0::2].set(jnp.sin(pos[:, 0::2]))
        pos = pos.at[:, 1::2].set(jnp.cos(pos[:, 1::2]))
        return np.asarray(pos)


@functools.lru_cache(maxsize=None)
def _sc_gather_fn(batch: int, sent: int, size: int, h: int, nsplit: int):
    ss = sent // nsplit
    n_rows = batch * ss
    bpw = n_rows // _NW
    nchunk = bpw // _CHUNK
    wpb = ss // bpw
    assert wpb * bpw == ss and wpb * batch == _NW
    mesh = plsc.VectorSubcoreMesh(core_axis_name="c", subcore_axis_name="s")

    @functools.partial(
        pl.kernel,
        mesh=mesh,
        out_type=jax.ShapeDtypeStruct((batch, ss, size), jnp.float32),
        scratch_types=[
            pltpu.VMEM((nchunk, _CHUNK), jnp.int32),
            pltpu.VMEM((2, _CHUNK, size), jnp.float32),
            pltpu.SemaphoreType.DMA,
            pltpu.SemaphoreType.DMA,
        ],
    )
    def gather_kernel(table_hbm, idx_hbm, out_hbm, idx_v, rows_v, sem0, sem1):
        wid = lax.axis_index("s") * _NC + lax.axis_index("c")
        b = wid // wpb
        col0 = (wid % wpb) * bpw
        for c in range(nchunk):
            pltpu.sync_copy(
                idx_hbm.at[b, pl.ds(h * ss + col0 + c * _CHUNK, _CHUNK)],
                idx_v.at[c],
            )

        def start(c, buf, sem):
            return pltpu.async_copy(
                table_hbm.at[idx_v.at[c]],
                rows_v.at[buf],
                sem,
            )

        sems = (sem0, sem1)
        cps = [start(0, 0, sem0)]
        if nchunk > 1:
            cps.append(start(1, 1, sem1))
        for c in range(nchunk):
            buf = c % 2
            cps[c].wait()
            pltpu.sync_copy(
                rows_v.at[buf],
                out_hbm.at[b, pl.ds(col0 + c * _CHUNK, _CHUNK)],
            )
            if c + 2 < nchunk:
                cps.append(start(c + 2, buf, sems[buf]))

    return gather_kernel


@functools.lru_cache(maxsize=None)
def _tc_ln_fn(batch: int, sent: int, size: int, s_blk: int, h: int,
              nsplit: int):
    ss = sent // nsplit
    seg_boundary = sent // 2 + 1
    spb = ss // s_blk
    j0 = h * spb
    grid = (spb, batch)
    alias = h > 0
    all_seg0 = (h + 1) * ss <= seg_boundary

    def body(*refs):
        if alias:
            refs = refs[1:]
        g_ref, seg_ref, pos_ref, gam_ref, bet_ref, o_ref = refs
        if all_seg0:
            segv = seg_ref[0]
        else:
            j = pl.program_id(0)
            rows = (h * ss + j * s_blk
                    + lax.broadcasted_iota(jnp.int32, (s_blk, 1), 0))
            segv = jnp.where(rows >= seg_boundary, seg_ref[1], seg_ref[0])
        x = g_ref[0] + segv + pos_ref[...]
        mean = jnp.mean(x, axis=-1, keepdims=True)
        xc = x - mean
        var = jnp.mean(xc * xc, axis=-1, keepdims=True)
        y = xc * lax.rsqrt(var + 1e-5)
        o_ref[0] = y * gam_ref[...] + bet_ref[...]

    in_specs = [
        pl.BlockSpec((1, s_blk, size), lambda j, b: (b, j, 0)),
        pl.BlockSpec((8, size), lambda j, b: (0, 0)),
        pl.BlockSpec((s_blk, size), lambda j, b: (j0 + j, 0)),
        pl.BlockSpec((size,), lambda j, b: (0,)),
        pl.BlockSpec((size,), lambda j, b: (0,)),
    ]
    kwargs = {}
    if alias:
        in_specs = [pl.BlockSpec(memory_space=pl.ANY)] + in_specs
        kwargs["input_output_aliases"] = {0: 0}

    return pl.pallas_call(
        body,
        grid=grid,
        in_specs=in_specs,
        out_specs=pl.BlockSpec((1, s_blk, size), lambda j, b: (b, j0 + j, 0)),
        out_shape=jax.ShapeDtypeStruct((batch, sent, size), jnp.float32),
        **kwargs,
    )


def kernel(input_tensor, token_table, segment_table, ln_gamma, ln_beta):
    batch, sent = input_tensor.shape
    size = token_table.shape[1]
    nsplit = 2 if (sent // 2) % (_NW // batch * _CHUNK) == 0 else 1

    parts = [
        _sc_gather_fn(batch, sent, size, h, nsplit)(token_table, input_tensor)
        for h in range(nsplit)
    ]

    pos = jnp.asarray(_pos_encoding(size, sent))
    s_blk = 1024

    out = _tc_ln_fn(batch, sent, size, s_blk, 0, nsplit)(
        parts[0], segment_table, pos, ln_gamma, ln_beta
    )
    for h in range(1, nsplit):
        out = _tc_ln_fn(batch, sent, size, s_blk, h, nsplit)(
            out, parts[h], segment_table, pos, ln_gamma, ln_beta
        )
    return out


# speedup vs baseline: 6.0736x; 1.0196x over previous
import functools

import jax
import jax.numpy as jnp
import numpy as np
from jax import lax
from jax.experimental import pallas as pl
from jax.experimental.pallas import tpu as pltpu
from jax.experimental.pallas import tpu_sc as plsc

_NC = 2
_NS = 16
_NW = _NC * _NS
_CHUNK = 128


@functools.lru_cache(maxsize=None)
def _pos_encoding(size: int, sent: int) -> np.ndarray:
    with jax.ensure_compile_time_eval():
        pos = jnp.arange(sent, dtype=jnp.float32)[:, None]
        d = 2.0 * jnp.arange(size, dtype=jnp.float32) / float(size)
        pos = pos / (1e4 ** d)
        pos = pos.at[:, 0::2].set(jnp.sin(pos[:, 0::2]))
        pos = pos.at[:, 1::2].set(jnp.cos(pos[:, 1::2]))
        return np.asarray(pos)


@functools.lru_cache(maxsize=None)
def _sc_gather_fn(batch: int, sent: int, size: int, h: int, nsplit: int):
    ss = sent // nsplit
    n_rows = batch * ss
    bpw = n_rows // _NW
    nchunk = bpw // _CHUNK
    wpb = ss // bpw
    assert wpb * bpw == ss and wpb * batch == _NW
    mesh = plsc.VectorSubcoreMesh(core_axis_name="c", subcore_axis_name="s")

    @functools.partial(
        pl.kernel,
        mesh=mesh,
        out_type=jax.ShapeDtypeStruct((batch, ss, size), jnp.float32),
        scratch_types=[
            pltpu.VMEM((nchunk, _CHUNK), jnp.int32),
            pltpu.VMEM((1, _CHUNK, size), jnp.float32),
            pltpu.SemaphoreType.DMA,
            pltpu.SemaphoreType.DMA,
        ],
    )
    def gather_kernel(table_hbm, idx_hbm, out_hbm, idx_v, rows_v, sem0, sem1):
        wid = lax.axis_index("s") * _NC + lax.axis_index("c")
        b = wid // wpb
        col0 = (wid % wpb) * bpw
        for c in range(nchunk):
            pltpu.sync_copy(
                idx_hbm.at[b, pl.ds(h * ss + col0 + c * _CHUNK, _CHUNK)],
                idx_v.at[c],
            )

        def start(c, buf, sem):
            return pltpu.async_copy(
                table_hbm.at[idx_v.at[c]],
                rows_v.at[buf],
                sem,
            )

        sems = (sem0, sem1)
        cps = [start(0, 0, sem0)]
        for c in range(nchunk):
            buf = 0
            cps[c].wait()
            pltpu.sync_copy(
                rows_v.at[buf],
                out_hbm.at[b, pl.ds(col0 + c * _CHUNK, _CHUNK)],
            )
            if c + 1 < nchunk:
                cps.append(start(c + 1, buf, sems[buf]))

    return gather_kernel


@functools.lru_cache(maxsize=None)
def _tc_ln_fn(batch: int, sent: int, size: int, s_blk: int, h: int,
              nsplit: int):
    ss = sent // nsplit
    seg_boundary = sent // 2 + 1
    spb = ss // s_blk
    j0 = h * spb
    grid = (spb, batch)
    alias = h > 0
    all_seg0 = (h + 1) * ss <= seg_boundary

    def body(*refs):
        if alias:
            refs = refs[1:]
        g_ref, seg_ref, pos_ref, gam_ref, bet_ref, o_ref = refs
        if all_seg0:
            segv = seg_ref[0]
        else:
            j = pl.program_id(0)
            rows = (h * ss + j * s_blk
                    + lax.broadcasted_iota(jnp.int32, (s_blk, 1), 0))
            segv = jnp.where(rows >= seg_boundary, seg_ref[1], seg_ref[0])
        x = g_ref[0] + segv + pos_ref[...]
        mean = jnp.mean(x, axis=-1, keepdims=True)
        xc = x - mean
        var = jnp.mean(xc * xc, axis=-1, keepdims=True)
        y = xc * lax.rsqrt(var + 1e-5)
        o_ref[0] = y * gam_ref[...] + bet_ref[...]

    in_specs = [
        pl.BlockSpec((1, s_blk, size), lambda j, b: (b, j, 0)),
        pl.BlockSpec((8, size), lambda j, b: (0, 0)),
        pl.BlockSpec((s_blk, size), lambda j, b: (j0 + j, 0)),
        pl.BlockSpec((size,), lambda j, b: (0,)),
        pl.BlockSpec((size,), lambda j, b: (0,)),
    ]
    kwargs = {}
    if alias:
        in_specs = [pl.BlockSpec(memory_space=pl.ANY)] + in_specs
        kwargs["input_output_aliases"] = {0: 0}

    return pl.pallas_call(
        body,
        grid=grid,
        in_specs=in_specs,
        out_specs=pl.BlockSpec((1, s_blk, size), lambda j, b: (b, j0 + j, 0)),
        out_shape=jax.ShapeDtypeStruct((batch, sent, size), jnp.float32),
        **kwargs,
    )


def kernel(input_tensor, token_table, segment_table, ln_gamma, ln_beta):
    batch, sent = input_tensor.shape
    size = token_table.shape[1]
    nsplit = 2 if (sent // 2) % (_NW // batch * _CHUNK) == 0 else 1

    parts = [
        _sc_gather_fn(batch, sent, size, h, nsplit)(token_table, input_tensor)
        for h in range(nsplit)
    ]

    pos = jnp.asarray(_pos_encoding(size, sent))
    s_blk = 1024

    out = _tc_ln_fn(batch, sent, size, s_blk, 0, nsplit)(
        parts[0], segment_table, pos, ln_gamma, ln_beta
    )
    for h in range(1, nsplit):
        out = _tc_ln_fn(batch, sent, size, s_blk, h, nsplit)(
            out, parts[h], segment_table, pos, ln_gamma, ln_beta
        )
    return out
